# baseline (speedup 1.0000x reference)
import jax
import jax.numpy as jnp
from jax.experimental import pallas as pl
from jax.experimental.pallas import tpu as pltpu

_R, _C = 2048, 4096
_NDMA = 8
_BAND = _R // _NDMA


def _copy_body(in_ref, out_ref, *sems):
    copies = []
    for i in range(_NDMA):
        cp = pltpu.make_async_copy(
            in_ref.at[pl.ds(i * _BAND, _BAND)],
            out_ref.at[pl.ds(i * _BAND, _BAND)],
            sems[i],
        )
        cp.start()
        copies.append(cp)
    for cp in copies:
        cp.wait()


def kernel(free_values):
    x = free_values.reshape(_R, _C)
    return pl.pallas_call(
        _copy_body,
        in_specs=[pl.BlockSpec(memory_space=pl.ANY)],
        out_specs=pl.BlockSpec(memory_space=pl.ANY),
        out_shape=jax.ShapeDtypeStruct((_R, _C), jnp.float32),
        scratch_shapes=[pltpu.SemaphoreType.DMA for _ in range(_NDMA)],
    )(x)


# speedup vs baseline: 17.1990x; 17.1990x over previous
import jax
import jax.numpy as jnp
from jax.experimental import pallas as pl
from jax.experimental.pallas import tpu as pltpu

_R, _C = 2048, 4096
_CHUNK = 256
_NCHUNK = _R // _CHUNK
_NBUF = 3


def _copy_body(in_ref, out_ref, *scratch):
    bufs = scratch[:_NBUF]
    isems = scratch[_NBUF:2 * _NBUF]
    osems = scratch[2 * _NBUF:]

    def rows(c):
        return pl.ds(c * _CHUNK, _CHUNK)

    ins = [None] * _NCHUNK
    outs = [None] * _NCHUNK
    for c in range(_NBUF):
        ins[c] = pltpu.make_async_copy(in_ref.at[rows(c)], bufs[c], isems[c])
        ins[c].start()
    for c in range(_NCHUNK):
        b = c % _NBUF
        if c >= _NBUF:
            outs[c - _NBUF].wait()
            ins[c] = pltpu.make_async_copy(in_ref.at[rows(c)], bufs[b], isems[b])
            ins[c].start()
        ins[c].wait()
        outs[c] = pltpu.make_async_copy(bufs[b], out_ref.at[rows(c)], osems[b])
        outs[c].start()
    for c in range(_NCHUNK - _NBUF, _NCHUNK):
        outs[c].wait()


def kernel(free_values):
    x = free_values.reshape(_R, _C)
    return pl.pallas_call(
        _copy_body,
        in_specs=[pl.BlockSpec(memory_space=pl.ANY)],
        out_specs=pl.BlockSpec(memory_space=pl.ANY),
        out_shape=jax.ShapeDtypeStruct((_R, _C), jnp.float32),
        scratch_shapes=(
            [pltpu.VMEM((_CHUNK, _C), jnp.float32) for _ in range(_NBUF)]
            + [pltpu.SemaphoreType.DMA for _ in range(2 * _NBUF)]
        ),
    )(x)


# speedup vs baseline: 19.0571x; 1.1080x over previous
import jax
import jax.numpy as jnp
from jax.experimental import pallas as pl
from jax.experimental.pallas import tpu as pltpu

_R, _C = 2048, 4096
_CHUNK = 128
_NCHUNK = _R // _CHUNK


def _copy_body(in_ref, out_ref, *scratch):
    bufs = scratch[:_NCHUNK]
    isems = scratch[_NCHUNK:2 * _NCHUNK]
    osems = scratch[2 * _NCHUNK:]

    def rows(c):
        return pl.ds(c * _CHUNK, _CHUNK)

    ins = []
    for c in range(_NCHUNK):
        cp = pltpu.make_async_copy(in_ref.at[rows(c)], bufs[c], isems[c])
        cp.start()
        ins.append(cp)
    outs = []
    for c in range(_NCHUNK):
        ins[c].wait()
        cp = pltpu.make_async_copy(bufs[c], out_ref.at[rows(c)], osems[c])
        cp.start()
        outs.append(cp)
    for cp in outs:
        cp.wait()


def kernel(free_values):
    x = free_values.reshape(_R, _C)
    return pl.pallas_call(
        _copy_body,
        in_specs=[pl.BlockSpec(memory_space=pl.ANY)],
        out_specs=pl.BlockSpec(memory_space=pl.ANY),
        out_shape=jax.ShapeDtypeStruct((_R, _C), jnp.float32),
        scratch_shapes=(
            [pltpu.VMEM((_CHUNK, _C), jnp.float32) for _ in range(_NCHUNK)]
            + [pltpu.SemaphoreType.DMA for _ in range(2 * _NCHUNK)]
        ),
    )(x)


# speedup vs baseline: 49.1102x; 2.5770x over previous
import jax
import jax.numpy as jnp
from jax.experimental import pallas as pl
from jax.experimental.pallas import tpu as pltpu

_R, _C = 2048, 4096
_CHUNK = 128
_NCHUNK = _R // _CHUNK


def _copy_body(in_ref, out_ref, *scratch):
    bufs = scratch[:_NCHUNK]
    isems = scratch[_NCHUNK:2 * _NCHUNK]
    osems = scratch[2 * _NCHUNK:]
    in2d = in_ref.reshape(_R, _C)

    def rows(c):
        return pl.ds(c * _CHUNK, _CHUNK)

    ins = []
    for c in range(_NCHUNK):
        cp = pltpu.make_async_copy(in2d.at[rows(c)], bufs[c], isems[c])
        cp.start()
        ins.append(cp)
    outs = []
    for c in range(_NCHUNK):
        ins[c].wait()
        cp = pltpu.make_async_copy(bufs[c], out_ref.at[rows(c)], osems[c])
        cp.start()
        outs.append(cp)
    for cp in outs:
        cp.wait()


def kernel(free_values):
    x = free_values.reshape(_R * _C // 128, 128)
    return pl.pallas_call(
        _copy_body,
        in_specs=[pl.BlockSpec(memory_space=pl.ANY)],
        out_specs=pl.BlockSpec(memory_space=pl.ANY),
        out_shape=jax.ShapeDtypeStruct((_R, _C), jnp.float32),
        scratch_shapes=(
            [pltpu.VMEM((_CHUNK, _C), jnp.float32) for _ in range(_NCHUNK)]
            + [pltpu.SemaphoreType.DMA for _ in range(2 * _NCHUNK)]
        ),
    )(x)
